# baseline (speedup 1.0000x reference)
import functools

import jax
import jax.numpy as jnp
from jax import lax
from jax.experimental import pallas as pl
from jax.experimental.pallas import tpu as pltpu
from jax.experimental.pallas import tpu_sc as plsc

N = 10000
IN_DIM = 16
CD = 3
H = 8
D = 64
L = 2
BLK = 100
NH = 3
NBK = 16
NW = 8
H2 = 32
NB = N // BLK
A = NH * H
NKEY = 128
RB = 1000
DH = D + CD
CB = 20


def _full(shape):
    nd = len(shape)
    return pl.BlockSpec(shape, lambda *_: (0,) * nd)


def _rank_body(key_ref, pos_ref):
    k2 = key_ref[0]
    iota_b = jax.lax.broadcasted_iota(jnp.int32, (1, 1, NKEY), 2)
    oh = (k2[:, :, None] == iota_b).astype(jnp.float32)
    hist = jnp.sum(oh, axis=1)
    r = jax.lax.broadcasted_iota(jnp.int32, (NB, NB), 0)
    c = jax.lax.broadcasted_iota(jnp.int32, (NB, NB), 1)
    ltri = (r >= c).astype(jnp.float32)
    cum_incl = jnp.dot(ltri, hist, preferred_element_type=jnp.float32)
    cum_excl = cum_incl - hist
    total = jnp.sum(hist, axis=0, keepdims=True)
    rb = jax.lax.broadcasted_iota(jnp.int32, (NKEY, NKEY), 0)
    cb = jax.lax.broadcasted_iota(jnp.int32, (NKEY, NKEY), 1)
    ustrict = (rb < cb).astype(jnp.float32)
    cdf_less = jnp.dot(total, ustrict, preferred_element_type=jnp.float32)
    term1 = jnp.sum(oh * cdf_less[None, :, :], axis=-1)
    term2 = jnp.sum(oh * cum_excl[:, None, :], axis=-1)
    eq = (k2[:, :, None] == k2[:, None, :]).astype(jnp.float32)
    rm = jax.lax.broadcasted_iota(jnp.int32, (1, BLK, BLK), 1)
    cm = jax.lax.broadcasted_iota(jnp.int32, (1, BLK, BLK), 2)
    lstrict = (rm > cm).astype(jnp.float32)
    term3 = jnp.sum(eq * lstrict, axis=-1)
    pos_ref[0] = (term1 + term2 + term3).astype(jnp.int32)


def _rank_call(key2):
    return pl.pallas_call(
        _rank_body,
        grid=(A,),
        in_specs=[pl.BlockSpec((1, NB, BLK), lambda a: (a, 0, 0))],
        out_specs=pl.BlockSpec((1, NB, BLK), lambda a: (a, 0, 0)),
        out_shape=jax.ShapeDtypeStruct((A, NB, BLK), jnp.int32),
    )(key2)


def _perm_sc_body(pos_hbm, out_hbm, pos_v, perm_v):
    nc = plsc.get_sparse_core_info().num_cores
    wid = lax.axis_index("s") * nc + lax.axis_index("c")

    @pl.when(wid < A)
    def _():
        pltpu.sync_copy(pos_hbm.at[wid], pos_v)

        def body(j, carry):
            idx = pos_v[pl.ds(j * 16, 16)]
            vals = lax.iota(jnp.int32, 16) + j * 16
            plsc.store_scatter(perm_v, [idx], vals)
            return carry

        lax.fori_loop(0, N // 16, body, 0)
        pltpu.sync_copy(perm_v, out_hbm.at[wid])


def _perm_sc_call(pos2):
    mesh = plsc.VectorSubcoreMesh(core_axis_name="c", subcore_axis_name="s")
    return functools.partial(
        pl.kernel, mesh=mesh,
        out_type=jax.ShapeDtypeStruct((A, N), jnp.int32),
        scratch_types=[pltpu.VMEM((N,), jnp.int32),
                       pltpu.VMEM((N,), jnp.int32)],
        compiler_params=pltpu.CompilerParams(needs_layout_passes=False),
    )(_perm_sc_body)(pos2)


def _enc_body(x_ref, w1_ref, b1_ref, w2_ref, b2_ref, o_ref):
    h = jnp.maximum(jnp.dot(x_ref[...], w1_ref[...],
                            preferred_element_type=jnp.float32) + b1_ref[...], 0.0)
    o_ref[...] = jnp.dot(h, w2_ref[...],
                         preferred_element_type=jnp.float32) + b2_ref[...]


def _enc_call(x, w1, b1, w2, b2):
    return pl.pallas_call(
        _enc_body,
        grid=(N // RB,),
        in_specs=[pl.BlockSpec((RB, IN_DIM), lambda i: (i, 0)),
                  _full((IN_DIM, D)), _full((1, D)), _full((D, D)), _full((1, D))],
        out_specs=pl.BlockSpec((RB, D), lambda i: (i, 0)),
        out_shape=jax.ShapeDtypeStruct((N, D), jnp.float32),
    )(x, w1, b1, w2, b2)


def _ln(x, g, b):
    m = jnp.mean(x, axis=-1, keepdims=True)
    v = jnp.mean((x - m) ** 2, axis=-1, keepdims=True)
    return (x - m) / jnp.sqrt(v + 1e-5) * g + b


FD = 2 * DH + D


def _pre_body(xc_ref, co_ref, pw1_ref, pb1_ref, pg1_ref, pbb1_ref, pw2_ref,
              pb2_ref, g1_ref, b1_ref, wqkv_ref, s_ref, qkvc_ref):
    co = co_ref[...]
    pe1 = _ln(jnp.dot(co, pw1_ref[...], preferred_element_type=jnp.float32)
              + pb1_ref[...], pg1_ref[...], pbb1_ref[...])
    pe = jnp.dot(jnp.maximum(pe1, 0.0), pw2_ref[...],
                 preferred_element_type=jnp.float32) + pb2_ref[...]
    xn = _ln(xc_ref[...] + pe, g1_ref[...], b1_ref[...])
    qkv = jnp.dot(xn, wqkv_ref[...], preferred_element_type=jnp.float32)
    for h in range(H):
        cc = co * s_ref[0, h]
        qkvc_ref[h] = jnp.concatenate(
            [qkv[:, h * D:(h + 1) * D], cc,
             qkv[:, H * D + h * D:H * D + (h + 1) * D], cc,
             qkv[:, 2 * H * D + h * D:2 * H * D + (h + 1) * D]], axis=-1)


def _pre_call(xc, coords, pw1, pb1, pg1, pbb1, pw2, pb2, g1, b1, wqkv, s):
    return pl.pallas_call(
        _pre_body,
        grid=(N // RB,),
        in_specs=[pl.BlockSpec((RB, D), lambda i: (i, 0)),
                  pl.BlockSpec((RB, CD), lambda i: (i, 0)),
                  _full((CD, D)), _full((1, D)), _full((1, D)), _full((1, D)),
                  _full((D, D)), _full((1, D)), _full((1, D)), _full((1, D)),
                  _full((D, 3 * H * D)), _full((1, H))],
        out_specs=pl.BlockSpec((H, RB, FD), lambda i: (0, i, 0)),
        out_shape=jax.ShapeDtypeStruct((H, N, FD), jnp.float32),
    )(xc, coords, pw1, pb1, pg1, pbb1, pw2, pb2, g1, b1, wqkv, s)


def _attn_body(s_ref, o_ref):
    scale = 1.0 / jnp.sqrt(float(DH))
    for b in range(CB):
        blk = s_ref[b * BLK:(b + 1) * BLK, :]
        q = blk[:, :DH]
        k = blk[:, DH:2 * DH]
        v = blk[:, 2 * DH:]
        sc = jax.lax.dot_general(q, k, (((1,), (1,)), ((), ())),
                                 preferred_element_type=jnp.float32) * scale
        m = jnp.max(sc, axis=-1, keepdims=True)
        e = jnp.exp(sc - m)
        o = jnp.dot(e, v, preferred_element_type=jnp.float32)
        o_ref[b * BLK:(b + 1) * BLK, :] = o / jnp.sum(e, axis=-1, keepdims=True)


def _attn_call(sorted_qkv):
    return pl.pallas_call(
        _attn_body,
        grid=(A * NB // CB,),
        in_specs=[pl.BlockSpec((CB * BLK, FD), lambda i: (i, 0))],
        out_specs=pl.BlockSpec((CB * BLK, D), lambda i: (i, 0)),
        out_shape=jax.ShapeDtypeStruct((A * N, D), jnp.float32),
    )(sorted_qkv)


def _post_body(xc_ref, ou_ref, wo_ref, wob_ref, g2_ref, b2_ref,
               fw1_ref, fb1_ref, fw2_ref, fb2_ref, o_ref):
    agg = wob_ref[...]
    for h in range(H):
        agg = agg + jnp.dot(ou_ref[h], wo_ref[h],
                            preferred_element_type=jnp.float32)
    xcn = xc_ref[...] + agg
    xn2 = _ln(xcn, g2_ref[...], b2_ref[...])
    ff = jnp.maximum(jnp.dot(xn2, fw1_ref[...],
                             preferred_element_type=jnp.float32) + fb1_ref[...], 0.0)
    ff = jnp.dot(ff, fw2_ref[...], preferred_element_type=jnp.float32) + fb2_ref[...]
    o_ref[...] = xcn + ff


def _post_call(xc, ou8, wo3, wob, g2, b2, fw1, fb1, fw2, fb2):
    return pl.pallas_call(
        _post_body,
        grid=(N // RB,),
        in_specs=[pl.BlockSpec((RB, D), lambda i: (i, 0)),
                  pl.BlockSpec((H, RB, D), lambda i: (0, i, 0)),
                  _full((H, D, D)), _full((1, D)), _full((1, D)), _full((1, D)),
                  _full((D, D)), _full((1, D)), _full((D, D)), _full((1, D))],
        out_specs=pl.BlockSpec((RB, D), lambda i: (i, 0)),
        out_shape=jax.ShapeDtypeStruct((N, D), jnp.float32),
    )(xc, ou8, wo3, wob, g2, b2, fw1, fb1, fw2, fb2)


def _head_body(h_ref, x1_ref, x2_ref, wout_ref, w0_ref, b0_ref, wh_ref,
               bh_ref, lng_ref, lnb_ref, wl_ref, bl_ref, o_ref):
    wo = wout_ref[...]
    z = (jnp.dot(h_ref[...], wo[:D], preferred_element_type=jnp.float32)
         + jnp.dot(x1_ref[...], wo[D:2 * D], preferred_element_type=jnp.float32)
         + jnp.dot(x2_ref[...], wo[2 * D:], preferred_element_type=jnp.float32))
    z = jnp.dot(z, w0_ref[...], preferred_element_type=jnp.float32) + b0_ref[...]
    z = jnp.tanh(_ln(z, lng_ref[0:1], lnb_ref[0:1]))
    for i in range(3):
        z = jnp.dot(z, wh_ref[i], preferred_element_type=jnp.float32) + bh_ref[i:i + 1]
        z = jnp.tanh(_ln(z, lng_ref[i + 1:i + 2], lnb_ref[i + 1:i + 2]))
    o_ref[...] = jnp.dot(z, wl_ref[...], preferred_element_type=jnp.float32) + bl_ref[...]


def _head_call(h, x1, x2, W_out, w0, b0, wh, bh, lng, lnb, wl, bl):
    return pl.pallas_call(
        _head_body,
        grid=(N // RB,),
        in_specs=[pl.BlockSpec((RB, D), lambda i: (i, 0)),
                  pl.BlockSpec((RB, D), lambda i: (i, 0)),
                  pl.BlockSpec((RB, D), lambda i: (i, 0)),
                  _full(((L + 1) * D, H2)), _full((H2, 256)), _full((1, 256)),
                  _full((3, 256, 256)), _full((3, 256)), _full((4, 256)),
                  _full((4, 256)), _full((256, H2)), _full((1, H2))],
        out_specs=pl.BlockSpec((RB, H2), lambda i: (i, 0)),
        out_shape=jax.ShapeDtypeStruct((N, H2), jnp.float32),
    )(h, x1, x2, W_out, w0, b0, wh, bh, lng, lnb, wl, bl)


def kernel(x, coords, edge_index, batch, bins, enc_w1, enc_b1, enc_w2, enc_b2,
           pe_w1, pe_b1, pe_g1, pe_bb1, pe_w2, pe_b2, n1_g, n1_b, wq, wk, wv,
           rpe_w, rpe_b, wo, wo_b, n2_g, n2_b, ff_w1, ff_b1, ff_w2, ff_b2,
           W_out, mlp_w0, mlp_b0, mlp_wh, mlp_bh, mlp_ln_g, mlp_ln_b,
           mlp_wl, mlp_bl):
    r2 = lambda a: a.reshape(1, -1)

    srt0 = jnp.argsort(coords[:, 0])
    srt1 = jnp.argsort(coords[:, 1])
    ar = jnp.arange(N, dtype=jnp.int32)
    rank0 = jnp.zeros(N, jnp.int32).at[srt0].set(ar)
    rank1 = jnp.zeros(N, jnp.int32).at[srt1].set(ar)
    bins_h = jnp.transpose(bins, (1, 0, 2)).reshape(2, A)
    bsz_e = jnp.ceil(N / bins_h[0])
    bsz_p = jnp.ceil(N / bins_h[1])
    be = rank0[None, :] // bsz_e[:, None] + 1.0
    bp = rank1[None, :] // bsz_p[:, None] + 1.0
    key = be.astype(jnp.int32) * 16 + bp.astype(jnp.int32)
    pos = _rank_call(key.reshape(A, NB, BLK)).reshape(A, N)
    perm = _perm_sc_call(pos)
    perm3 = perm.reshape(NH, H, N)
    pos3 = pos.reshape(NH, H, N)
    gidx = (perm3 + (jnp.arange(H, dtype=jnp.int32) * N)[None, :, None]).reshape(A * N)

    h0 = _enc_call(x, enc_w1, r2(enc_b1), enc_w2, r2(enc_b2))

    outs = [h0]
    xc = h0
    for l in range(L):
        rw = (jnp.sum(rpe_w[l], axis=0) + rpe_b[l]).reshape(H, D)
        s = jnp.mean(rw, axis=-1)
        wqkv = jnp.concatenate([wq[l], wk[l], wv[l]], axis=1)
        qkvc = _pre_call(
            xc, coords, pe_w1[l], r2(pe_b1[l]), r2(pe_g1[l]), r2(pe_bb1[l]),
            pe_w2[l], r2(pe_b2[l]), r2(n1_g[l]), r2(n1_b[l]), wqkv, r2(s))
        srt_qkv = qkvc.reshape(H * N, FD)[gidx]
        ob = _attn_call(srt_qkv).reshape(NH, H, N, D)
        ou8 = jnp.mean(jnp.take_along_axis(ob, pos3[..., None], axis=2), axis=0)
        xc = _post_call(xc, ou8, wo[l].reshape(H, D, D), r2(wo_b[l]),
                        r2(n2_g[l]), r2(n2_b[l]), ff_w1[l], r2(ff_b1[l]),
                        ff_w2[l], r2(ff_b2[l]))
        outs.append(xc)

    return _head_call(outs[0], outs[1], outs[2], W_out, mlp_w0, r2(mlp_b0),
                      mlp_wh, mlp_bh, mlp_ln_g, mlp_ln_b, mlp_wl, r2(mlp_bl))


# speedup vs baseline: 1.1705x; 1.1705x over previous
import functools

import jax
import jax.numpy as jnp
from jax import lax
from jax.experimental import pallas as pl
from jax.experimental.pallas import tpu as pltpu
from jax.experimental.pallas import tpu_sc as plsc

N = 10000
IN_DIM = 16
CD = 3
H = 8
D = 64
L = 2
BLK = 100
NH = 3
NBK = 16
NW = 8
H2 = 32
NB = N // BLK
A = NH * H
NKEY = 128
RB = 1000
DH = D + CD
CB = 10


def _full(shape):
    nd = len(shape)
    return pl.BlockSpec(shape, lambda *_: (0,) * nd)


def _rank_body(key_ref, pos_ref):
    k2 = key_ref[0]
    iota_b = jax.lax.broadcasted_iota(jnp.int32, (1, 1, NKEY), 2)
    oh = (k2[:, :, None] == iota_b).astype(jnp.float32)
    hist = jnp.sum(oh, axis=1)
    r = jax.lax.broadcasted_iota(jnp.int32, (NB, NB), 0)
    c = jax.lax.broadcasted_iota(jnp.int32, (NB, NB), 1)
    ltri = (r >= c).astype(jnp.float32)
    cum_incl = jnp.dot(ltri, hist, preferred_element_type=jnp.float32)
    cum_excl = cum_incl - hist
    total = jnp.sum(hist, axis=0, keepdims=True)
    rb = jax.lax.broadcasted_iota(jnp.int32, (NKEY, NKEY), 0)
    cb = jax.lax.broadcasted_iota(jnp.int32, (NKEY, NKEY), 1)
    ustrict = (rb < cb).astype(jnp.float32)
    cdf_less = jnp.dot(total, ustrict, preferred_element_type=jnp.float32)
    term1 = jnp.sum(oh * cdf_less[None, :, :], axis=-1)
    term2 = jnp.sum(oh * cum_excl[:, None, :], axis=-1)
    eq = (k2[:, :, None] == k2[:, None, :]).astype(jnp.float32)
    rm = jax.lax.broadcasted_iota(jnp.int32, (1, BLK, BLK), 1)
    cm = jax.lax.broadcasted_iota(jnp.int32, (1, BLK, BLK), 2)
    lstrict = (rm > cm).astype(jnp.float32)
    term3 = jnp.sum(eq * lstrict, axis=-1)
    pos_ref[0] = (term1 + term2 + term3).astype(jnp.int32)


def _rank_call(key2):
    return pl.pallas_call(
        _rank_body,
        grid=(A,),
        in_specs=[pl.BlockSpec((1, NB, BLK), lambda a: (a, 0, 0))],
        out_specs=pl.BlockSpec((1, NB, BLK), lambda a: (a, 0, 0)),
        out_shape=jax.ShapeDtypeStruct((A, NB, BLK), jnp.int32),
    )(key2)


def _perm_sc_body(pos_hbm, out_hbm, pos_v, perm_v):
    nc = plsc.get_sparse_core_info().num_cores
    wid = lax.axis_index("s") * nc + lax.axis_index("c")

    @pl.when(wid < A)
    def _():
        pltpu.sync_copy(pos_hbm.at[wid], pos_v)

        def body(j, carry):
            idx = pos_v[pl.ds(j * 16, 16)]
            vals = lax.iota(jnp.int32, 16) + j * 16
            plsc.store_scatter(perm_v, [idx], vals)
            return carry

        lax.fori_loop(0, N // 16, body, 0)
        pltpu.sync_copy(perm_v, out_hbm.at[wid])


def _perm_sc_call(pos2):
    mesh = plsc.VectorSubcoreMesh(core_axis_name="c", subcore_axis_name="s")
    return functools.partial(
        pl.kernel, mesh=mesh,
        out_type=jax.ShapeDtypeStruct((A, N), jnp.int32),
        scratch_types=[pltpu.VMEM((N,), jnp.int32),
                       pltpu.VMEM((N,), jnp.int32)],
        compiler_params=pltpu.CompilerParams(needs_layout_passes=False),
    )(_perm_sc_body)(pos2)


def _enc_body(x_ref, w1_ref, b1_ref, w2_ref, b2_ref, o_ref):
    h = jnp.maximum(jnp.dot(x_ref[...], w1_ref[...],
                            preferred_element_type=jnp.float32) + b1_ref[...], 0.0)
    o_ref[...] = jnp.dot(h, w2_ref[...],
                         preferred_element_type=jnp.float32) + b2_ref[...]


def _enc_call(x, w1, b1, w2, b2):
    return pl.pallas_call(
        _enc_body,
        grid=(N // RB,),
        in_specs=[pl.BlockSpec((RB, IN_DIM), lambda i: (i, 0)),
                  _full((IN_DIM, D)), _full((1, D)), _full((D, D)), _full((1, D))],
        out_specs=pl.BlockSpec((RB, D), lambda i: (i, 0)),
        out_shape=jax.ShapeDtypeStruct((N, D), jnp.float32),
    )(x, w1, b1, w2, b2)


def _ln(x, g, b):
    m = jnp.mean(x, axis=-1, keepdims=True)
    v = jnp.mean((x - m) ** 2, axis=-1, keepdims=True)
    return (x - m) / jnp.sqrt(v + 1e-5) * g + b


FD = 2 * DH + D


def _pre_body(xc_ref, co_ref, pw1_ref, pb1_ref, pg1_ref, pbb1_ref, pw2_ref,
              pb2_ref, g1_ref, b1_ref, wqkv_ref, s_ref, qkvc_ref):
    co = co_ref[...]
    pe1 = _ln(jnp.dot(co, pw1_ref[...], preferred_element_type=jnp.float32)
              + pb1_ref[...], pg1_ref[...], pbb1_ref[...])
    pe = jnp.dot(jnp.maximum(pe1, 0.0), pw2_ref[...],
                 preferred_element_type=jnp.float32) + pb2_ref[...]
    xn = _ln(xc_ref[...] + pe, g1_ref[...], b1_ref[...])
    qkv = jnp.dot(xn, wqkv_ref[...], preferred_element_type=jnp.float32)
    for h in range(H):
        cc = co * s_ref[0, h]
        qkvc_ref[h] = jnp.concatenate(
            [qkv[:, h * D:(h + 1) * D], cc,
             qkv[:, H * D + h * D:H * D + (h + 1) * D], cc,
             qkv[:, 2 * H * D + h * D:2 * H * D + (h + 1) * D]], axis=-1)


def _pre_call(xc, coords, pw1, pb1, pg1, pbb1, pw2, pb2, g1, b1, wqkv, s):
    return pl.pallas_call(
        _pre_body,
        grid=(N // RB,),
        in_specs=[pl.BlockSpec((RB, D), lambda i: (i, 0)),
                  pl.BlockSpec((RB, CD), lambda i: (i, 0)),
                  _full((CD, D)), _full((1, D)), _full((1, D)), _full((1, D)),
                  _full((D, D)), _full((1, D)), _full((1, D)), _full((1, D)),
                  _full((D, 3 * H * D)), _full((1, H))],
        out_specs=pl.BlockSpec((H, RB, FD), lambda i: (0, i, 0)),
        out_shape=jax.ShapeDtypeStruct((H, N, FD), jnp.float32),
    )(xc, coords, pw1, pb1, pg1, pbb1, pw2, pb2, g1, b1, wqkv, s)


def _attn_body(s_ref, o_ref):
    scale = 1.0 / jnp.sqrt(float(DH))
    for b in range(CB):
        blk = s_ref[b * BLK:(b + 1) * BLK, :]
        q = blk[:, :DH]
        k = blk[:, DH:2 * DH]
        v = blk[:, 2 * DH:]
        sc = jax.lax.dot_general(q, k, (((1,), (1,)), ((), ())),
                                 preferred_element_type=jnp.float32) * scale
        m = jnp.max(sc, axis=-1, keepdims=True)
        e = jnp.exp(sc - m)
        o = jnp.dot(e, v, preferred_element_type=jnp.float32)
        o_ref[b * BLK:(b + 1) * BLK, :] = o / jnp.sum(e, axis=-1, keepdims=True)


def _attn_call(sorted_qkv):
    return pl.pallas_call(
        _attn_body,
        grid=(A * NB // CB,),
        in_specs=[pl.BlockSpec((CB * BLK, FD), lambda i: (i, 0))],
        out_specs=pl.BlockSpec((CB * BLK, D), lambda i: (i, 0)),
        out_shape=jax.ShapeDtypeStruct((A * N, D), jnp.float32),
    )(sorted_qkv)


def _post_body(xc_ref, ou_ref, wo_ref, wob_ref, g2_ref, b2_ref,
               fw1_ref, fb1_ref, fw2_ref, fb2_ref, o_ref):
    agg = wob_ref[...]
    for h in range(H):
        agg = agg + jnp.dot(ou_ref[h], wo_ref[h],
                            preferred_element_type=jnp.float32)
    xcn = xc_ref[...] + agg
    xn2 = _ln(xcn, g2_ref[...], b2_ref[...])
    ff = jnp.maximum(jnp.dot(xn2, fw1_ref[...],
                             preferred_element_type=jnp.float32) + fb1_ref[...], 0.0)
    ff = jnp.dot(ff, fw2_ref[...], preferred_element_type=jnp.float32) + fb2_ref[...]
    o_ref[...] = xcn + ff


def _post_call(xc, ou8, wo3, wob, g2, b2, fw1, fb1, fw2, fb2):
    return pl.pallas_call(
        _post_body,
        grid=(N // RB,),
        in_specs=[pl.BlockSpec((RB, D), lambda i: (i, 0)),
                  pl.BlockSpec((H, RB, D), lambda i: (0, i, 0)),
                  _full((H, D, D)), _full((1, D)), _full((1, D)), _full((1, D)),
                  _full((D, D)), _full((1, D)), _full((D, D)), _full((1, D))],
        out_specs=pl.BlockSpec((RB, D), lambda i: (i, 0)),
        out_shape=jax.ShapeDtypeStruct((N, D), jnp.float32),
    )(xc, ou8, wo3, wob, g2, b2, fw1, fb1, fw2, fb2)


def _head_body(h_ref, x1_ref, x2_ref, wout_ref, w0_ref, b0_ref, wh_ref,
               bh_ref, lng_ref, lnb_ref, wl_ref, bl_ref, o_ref):
    wo = wout_ref[...]
    z = (jnp.dot(h_ref[...], wo[:D], preferred_element_type=jnp.float32)
         + jnp.dot(x1_ref[...], wo[D:2 * D], preferred_element_type=jnp.float32)
         + jnp.dot(x2_ref[...], wo[2 * D:], preferred_element_type=jnp.float32))
    z = jnp.dot(z, w0_ref[...], preferred_element_type=jnp.float32) + b0_ref[...]
    z = jnp.tanh(_ln(z, lng_ref[0:1], lnb_ref[0:1]))
    for i in range(3):
        z = jnp.dot(z, wh_ref[i], preferred_element_type=jnp.float32) + bh_ref[i:i + 1]
        z = jnp.tanh(_ln(z, lng_ref[i + 1:i + 2], lnb_ref[i + 1:i + 2]))
    o_ref[...] = jnp.dot(z, wl_ref[...], preferred_element_type=jnp.float32) + bl_ref[...]


def _head_call(h, x1, x2, W_out, w0, b0, wh, bh, lng, lnb, wl, bl):
    return pl.pallas_call(
        _head_body,
        grid=(N // RB,),
        in_specs=[pl.BlockSpec((RB, D), lambda i: (i, 0)),
                  pl.BlockSpec((RB, D), lambda i: (i, 0)),
                  pl.BlockSpec((RB, D), lambda i: (i, 0)),
                  _full(((L + 1) * D, H2)), _full((H2, 256)), _full((1, 256)),
                  _full((3, 256, 256)), _full((3, 256)), _full((4, 256)),
                  _full((4, 256)), _full((256, H2)), _full((1, H2))],
        out_specs=pl.BlockSpec((RB, H2), lambda i: (i, 0)),
        out_shape=jax.ShapeDtypeStruct((N, H2), jnp.float32),
    )(h, x1, x2, W_out, w0, b0, wh, bh, lng, lnb, wl, bl)


def kernel(x, coords, edge_index, batch, bins, enc_w1, enc_b1, enc_w2, enc_b2,
           pe_w1, pe_b1, pe_g1, pe_bb1, pe_w2, pe_b2, n1_g, n1_b, wq, wk, wv,
           rpe_w, rpe_b, wo, wo_b, n2_g, n2_b, ff_w1, ff_b1, ff_w2, ff_b2,
           W_out, mlp_w0, mlp_b0, mlp_wh, mlp_bh, mlp_ln_g, mlp_ln_b,
           mlp_wl, mlp_bl):
    r2 = lambda a: a.reshape(1, -1)

    srt0 = jnp.argsort(coords[:, 0])
    srt1 = jnp.argsort(coords[:, 1])
    ar = jnp.arange(N, dtype=jnp.int32)
    rank0 = jnp.zeros(N, jnp.int32).at[srt0].set(ar)
    rank1 = jnp.zeros(N, jnp.int32).at[srt1].set(ar)
    bins_h = jnp.transpose(bins, (1, 0, 2)).reshape(2, A)
    bsz_e = jnp.ceil(N / bins_h[0])
    bsz_p = jnp.ceil(N / bins_h[1])
    be = rank0[None, :] // bsz_e[:, None] + 1.0
    bp = rank1[None, :] // bsz_p[:, None] + 1.0
    key = be.astype(jnp.int32) * 16 + bp.astype(jnp.int32)
    pos = _rank_call(key.reshape(A, NB, BLK)).reshape(A, N)
    perm = _perm_sc_call(pos)
    perm3 = perm.reshape(NH, H, N)
    pos3 = pos.reshape(NH, H, N)
    gidx = (perm3 + (jnp.arange(H, dtype=jnp.int32) * N)[None, :, None]).reshape(A * N)

    h0 = _enc_call(x, enc_w1, r2(enc_b1), enc_w2, r2(enc_b2))

    outs = [h0]
    xc = h0
    for l in range(L):
        rw = (jnp.sum(rpe_w[l], axis=0) + rpe_b[l]).reshape(H, D)
        s = jnp.mean(rw, axis=-1)
        wqkv = jnp.concatenate([wq[l], wk[l], wv[l]], axis=1)
        qkvc = _pre_call(
            xc, coords, pe_w1[l], r2(pe_b1[l]), r2(pe_g1[l]), r2(pe_bb1[l]),
            pe_w2[l], r2(pe_b2[l]), r2(n1_g[l]), r2(n1_b[l]), wqkv, r2(s))
        srt_qkv = qkvc.reshape(H * N, FD)[gidx]
        ob = _attn_call(srt_qkv).reshape(NH, H, N, D)
        ou8 = jnp.mean(jnp.take_along_axis(ob, pos3[..., None], axis=2), axis=0)
        xc = _post_call(xc, ou8, wo[l].reshape(H, D, D), r2(wo_b[l]),
                        r2(n2_g[l]), r2(n2_b[l]), ff_w1[l], r2(ff_b1[l]),
                        ff_w2[l], r2(ff_b2[l]))
        outs.append(xc)

    return _head_call(outs[0], outs[1], outs[2], W_out, mlp_w0, r2(mlp_b0),
                      mlp_wh, mlp_bh, mlp_ln_g, mlp_ln_b, mlp_wl, r2(mlp_bl))
